# Initial kernel scaffold; baseline (speedup 1.0000x reference)
import jax
import jax.numpy as jnp
from jax import lax
from jax.experimental import pallas as pl
from jax.experimental.pallas import tpu as pltpu
from jax.experimental.pallas import tpu_sc as plsc

N_PAD = 50176
ROWS = N_PAD // 128
E_TOTAL = 1600000
BLK = 1280
NB = BLK // 128
NBLOCKS = E_TOTAL // BLK
NW = 32
PER_TILE_SLICE = N_PAD // 16


def _node_xy_body(v_ref, th_ref, x_ref, y_ref):
    m = jnp.abs(v_ref[...])
    th = th_ref[...]
    x_ref[...] = m * jnp.cos(th)
    y_ref[...] = m * jnp.sin(th)


def _node_xy(v2, th2):
    return pl.pallas_call(
        _node_xy_body,
        out_shape=(
            jax.ShapeDtypeStruct((ROWS, 128), jnp.float32),
            jax.ShapeDtypeStruct((ROWS, 128), jnp.float32),
        ),
    )(v2, th2)


def _edge_body(x_hbm, y_hbm, from_hbm, to_hbm, attr_hbm, pact_hbm, prea_hbm,
               xv, yv, fiv, tiv, attrv, actv, reav, acc_act, acc_rea):
    cid = lax.axis_index("c")
    sid = lax.axis_index("s")
    wid = cid * 16 + sid

    def _z(i, c):
        xv[pl.ds(i * 16, 16)] = jnp.zeros((16,), jnp.float32)
        return c
    lax.fori_loop(0, PER_TILE_SLICE // 16, _z, 0)
    pltpu.sync_copy(xv.at[pl.ds(0, PER_TILE_SLICE)],
                    acc_act.at[pl.ds(sid * PER_TILE_SLICE, PER_TILE_SLICE)])
    pltpu.sync_copy(xv.at[pl.ds(0, PER_TILE_SLICE)],
                    acc_rea.at[pl.ds(sid * PER_TILE_SLICE, PER_TILE_SLICE)])

    pltpu.sync_copy(x_hbm, xv)
    pltpu.sync_copy(y_hbm, yv)
    plsc.subcore_barrier()

    lanes = lax.iota(jnp.int32, 16)

    def _process_block(bid):
        pltpu.sync_copy(from_hbm.at[pl.ds(bid * NB, NB)], fiv)
        pltpu.sync_copy(to_hbm.at[pl.ds(bid * BLK, BLK)], tiv)
        pltpu.sync_copy(attr_hbm.at[pl.ds(bid * BLK * 4, BLK * 4)], attrv)

        def _row(j, c):
            for i2 in range(8):
                fi = fiv[j, pl.ds(i2 * 16, 16)]
                ti = tiv[pl.ds(j * 128 + i2 * 16, 16)]
                xf = plsc.load_gather(xv, [fi])
                yf = plsc.load_gather(yv, [fi])
                xt = plsc.load_gather(xv, [ti])
                yt = plsc.load_gather(yv, [ti])
                ai = (j * 128 + i2 * 16 + lanes) * 4
                a0 = plsc.load_gather(attrv, [ai])
                a1 = plsc.load_gather(attrv, [ai + 1])
                p = xf * xt + yf * yt
                q = yf * xt - xf * yt
                actv[j, pl.ds(i2 * 16, 16)] = a0 * p + a1 * q
                reav[j, pl.ds(i2 * 16, 16)] = a0 * q - a1 * p
            return c
        lax.fori_loop(0, NB, _row, 0)

        for j in range(NB):
            pltpu.sync_copy(actv.at[j], acc_act.at[fiv.at[j]], add=True)
            pltpu.sync_copy(reav.at[j], acc_rea.at[fiv.at[j]], add=True)

    for k in range(40):
        bid = wid + NW * k
        if k < 39:
            _process_block(bid)
        else:
            @pl.when(bid < NBLOCKS)
            def _():
                _process_block(bid)

    plsc.subcore_barrier()

    sl = pl.ds(sid * PER_TILE_SLICE, PER_TILE_SLICE)
    pltpu.sync_copy(acc_act.at[sl], xv.at[pl.ds(0, PER_TILE_SLICE)])
    pltpu.sync_copy(xv.at[pl.ds(0, PER_TILE_SLICE)], pact_hbm.at[cid, sl])
    pltpu.sync_copy(acc_rea.at[sl], yv.at[pl.ds(0, PER_TILE_SLICE)])
    pltpu.sync_copy(yv.at[pl.ds(0, PER_TILE_SLICE)], prea_hbm.at[cid, sl])


def _edge_accumulate(x1, y1, from2, to1, attr1):
    mesh = plsc.VectorSubcoreMesh(core_axis_name="c", subcore_axis_name="s")
    f = pl.kernel(
        _edge_body,
        out_type=(
            jax.ShapeDtypeStruct((2, N_PAD), jnp.float32),
            jax.ShapeDtypeStruct((2, N_PAD), jnp.float32),
        ),
        mesh=mesh,
        scratch_types=[
            pltpu.VMEM((N_PAD,), jnp.float32),
            pltpu.VMEM((N_PAD,), jnp.float32),
            pltpu.VMEM((NB, 128), jnp.int32),
            pltpu.VMEM((BLK,), jnp.int32),
            pltpu.VMEM((BLK * 4,), jnp.float32),
            pltpu.VMEM((NB, 128), jnp.float32),
            pltpu.VMEM((NB, 128), jnp.float32),
            pltpu.VMEM_SHARED((N_PAD,), jnp.float32),
            pltpu.VMEM_SHARED((N_PAD,), jnp.float32),
        ],
    )
    return f(x1, y1, from2, to1, attr1)


def _loss_body(o0_ref, o1_ref, pact_ref, prea_ref, out_ref):
    a = pact_ref[0] + pact_ref[1]
    r = prea_ref[0] + prea_ref[1]
    out_ref[0, 0] = jnp.sum(jnp.abs(o0_ref[...] - a) + jnp.abs(o1_ref[...] - r))


def _final_loss(o0, o1, pact, prea):
    return pl.pallas_call(
        _loss_body,
        out_shape=jax.ShapeDtypeStruct((1, 1), jnp.float32),
        out_specs=pl.BlockSpec(memory_space=pltpu.SMEM),
    )(o0, o1, pact, prea)


@jax.jit
def kernel(inputs, output, edges, attributes):
    del inputs
    n = output.shape[0]
    pad = N_PAD - n
    v2 = jnp.pad(output[:, 2], (0, pad)).reshape(ROWS, 128)
    th2 = jnp.pad(output[:, 3], (0, pad)).reshape(ROWS, 128)
    x2, y2 = _node_xy(v2, th2)

    from2 = edges[0].astype(jnp.int32).reshape(E_TOTAL // 128, 128)
    to1 = edges[1].astype(jnp.int32)
    attr1 = attributes.reshape(-1)
    pact, prea = _edge_accumulate(x2.reshape(-1), y2.reshape(-1),
                                  from2, to1, attr1)

    o0 = jnp.pad(output[:, 0], (0, pad)).reshape(ROWS, 128)
    o1 = jnp.pad(output[:, 1], (0, pad)).reshape(ROWS, 128)
    loss = _final_loss(o0, o1, pact.reshape(2, ROWS, 128),
                       prea.reshape(2, ROWS, 128))
    return loss[0, 0]


# speedup vs baseline: 4.5398x; 4.5398x over previous
import jax
import jax.numpy as jnp
from jax import lax
from jax.experimental import pallas as pl
from jax.experimental.pallas import tpu as pltpu
from jax.experimental.pallas import tpu_sc as plsc

N_PAD = 50176
ROWS = N_PAD // 128
E_TOTAL = 1600000
BLK = 2048
NFULL = E_TOTAL // BLK
TAIL_BID = NFULL
TAIL_VALID_FROM = NFULL * BLK - (E_TOTAL - BLK)
NW = 32
KMAX = (NFULL + 1 + NW - 1) // NW
PER_TILE_SLICE = N_PAD // 16
DUMP_NODE = N_PAD - 1


def _node_xy_body(v_ref, th_ref, x_ref, y_ref):
    m = jnp.abs(v_ref[...])
    th = th_ref[...]
    x_ref[...] = m * jnp.cos(th)
    y_ref[...] = m * jnp.sin(th)


def _node_xy(v2, th2):
    return pl.pallas_call(
        _node_xy_body,
        out_shape=(
            jax.ShapeDtypeStruct((ROWS, 128), jnp.float32),
            jax.ShapeDtypeStruct((ROWS, 128), jnp.float32),
        ),
    )(v2, th2)


def _edge_body(x_hbm, y_hbm, from_hbm, to_hbm, attr_hbm,
               pact0_hbm, pact1_hbm, prea0_hbm, prea1_hbm,
               xv, yv, fiv, tiv, attrv, actv, reav, acc_act, acc_rea):
    cid = lax.axis_index("c")
    sid = lax.axis_index("s")
    wid = cid * 16 + sid

    def _z(i, c):
        xv[pl.ds(i * 16, 16)] = jnp.zeros((16,), jnp.float32)
        return c
    lax.fori_loop(0, PER_TILE_SLICE // 16, _z, 0)
    pltpu.sync_copy(xv.at[pl.ds(0, PER_TILE_SLICE)],
                    acc_act.at[pl.ds(sid * PER_TILE_SLICE, PER_TILE_SLICE)])
    pltpu.sync_copy(xv.at[pl.ds(0, PER_TILE_SLICE)],
                    acc_rea.at[pl.ds(sid * PER_TILE_SLICE, PER_TILE_SLICE)])

    pltpu.sync_copy(x_hbm, xv)
    pltpu.sync_copy(y_hbm, yv)
    plsc.subcore_barrier()

    lanes = lax.iota(jnp.int32, 16)

    def _process_block(base, tail):
        pltpu.sync_copy(from_hbm.at[pl.ds(base, BLK)], fiv)
        pltpu.sync_copy(to_hbm.at[pl.ds(base, BLK)], tiv)
        pltpu.sync_copy(attr_hbm.at[pl.ds(base * 4, BLK * 4)], attrv)

        def _vec(j, c):
            fi = fiv[pl.ds(j * 16, 16)]
            ti = tiv[pl.ds(j * 16, 16)]
            if tail:
                ok = (j * 16 + lanes) >= TAIL_VALID_FROM
                fi = jnp.where(ok, fi, DUMP_NODE)
                fiv[pl.ds(j * 16, 16)] = fi
            xf = plsc.load_gather(xv, [fi])
            yf = plsc.load_gather(yv, [fi])
            xt = plsc.load_gather(xv, [ti])
            yt = plsc.load_gather(yv, [ti])
            ai = (j * 16 + lanes) * 4
            a0 = plsc.load_gather(attrv, [ai])
            a1 = plsc.load_gather(attrv, [ai + 1])
            p = xf * xt + yf * yt
            q = yf * xt - xf * yt
            act = a0 * p + a1 * q
            rea = a0 * q - a1 * p
            if tail:
                act = jnp.where(ok, act, 0.0)
                rea = jnp.where(ok, rea, 0.0)
            actv[pl.ds(j * 16, 16)] = act
            reav[pl.ds(j * 16, 16)] = rea
            return c
        lax.fori_loop(0, BLK // 16, _vec, 0)

        pltpu.sync_copy(actv, acc_act.at[fiv], add=True)
        pltpu.sync_copy(reav, acc_rea.at[fiv], add=True)

    for k in range(KMAX):
        bid = wid + NW * k
        if k < KMAX - 1:
            _process_block(bid * BLK, tail=False)
        else:
            @pl.when(bid < NFULL)
            def _():
                _process_block(bid * BLK, tail=False)

            @pl.when(bid == TAIL_BID)
            def _():
                _process_block(E_TOTAL - BLK, tail=True)

    plsc.subcore_barrier()

    sl = pl.ds(sid * PER_TILE_SLICE, PER_TILE_SLICE)
    tsl = pl.ds(0, PER_TILE_SLICE)
    pltpu.sync_copy(acc_act.at[sl], xv.at[tsl])
    pltpu.sync_copy(acc_rea.at[sl], yv.at[tsl])

    @pl.when(cid == 0)
    def _():
        pltpu.sync_copy(xv.at[tsl], pact0_hbm.at[sl])
        pltpu.sync_copy(yv.at[tsl], prea0_hbm.at[sl])

    @pl.when(cid == 1)
    def _():
        pltpu.sync_copy(xv.at[tsl], pact1_hbm.at[sl])
        pltpu.sync_copy(yv.at[tsl], prea1_hbm.at[sl])


def _edge_accumulate(x1, y1, from1, to1, attr1):
    mesh = plsc.VectorSubcoreMesh(core_axis_name="c", subcore_axis_name="s")
    f = pl.kernel(
        _edge_body,
        out_type=tuple(
            jax.ShapeDtypeStruct((N_PAD,), jnp.float32) for _ in range(4)),
        mesh=mesh,
        compiler_params=pltpu.CompilerParams(needs_layout_passes=False),
        scratch_types=[
            pltpu.VMEM((N_PAD,), jnp.float32),
            pltpu.VMEM((N_PAD,), jnp.float32),
            pltpu.VMEM((BLK,), jnp.int32),
            pltpu.VMEM((BLK,), jnp.int32),
            pltpu.VMEM((BLK * 4,), jnp.float32),
            pltpu.VMEM((BLK,), jnp.float32),
            pltpu.VMEM((BLK,), jnp.float32),
            pltpu.VMEM_SHARED((N_PAD,), jnp.float32),
            pltpu.VMEM_SHARED((N_PAD,), jnp.float32),
        ],
    )
    return f(x1, y1, from1, to1, attr1)


def _loss_body(o0_ref, o1_ref, a0_ref, a1_ref, r0_ref, r1_ref, out_ref):
    a = a0_ref[...] + a1_ref[...]
    r = r0_ref[...] + r1_ref[...]
    out_ref[0, 0] = jnp.sum(jnp.abs(o0_ref[...] - a) + jnp.abs(o1_ref[...] - r))


def _final_loss(o0, o1, a0, a1, r0, r1):
    return pl.pallas_call(
        _loss_body,
        out_shape=jax.ShapeDtypeStruct((1, 1), jnp.float32),
        out_specs=pl.BlockSpec(memory_space=pltpu.SMEM),
    )(o0, o1, a0, a1, r0, r1)


@jax.jit
def kernel(inputs, output, edges, attributes):
    del inputs
    n = output.shape[0]
    pad = N_PAD - n
    v2 = jnp.pad(output[:, 2], (0, pad)).reshape(ROWS, 128)
    th2 = jnp.pad(output[:, 3], (0, pad)).reshape(ROWS, 128)
    x2, y2 = _node_xy(v2, th2)

    from1 = edges[0].astype(jnp.int32)
    to1 = edges[1].astype(jnp.int32)
    attr1 = attributes.reshape(-1)
    pa0, pa1, pr0, pr1 = _edge_accumulate(
        x2.reshape(-1), y2.reshape(-1), from1, to1, attr1)

    o0 = jnp.pad(output[:, 0], (0, pad)).reshape(ROWS, 128)
    o1 = jnp.pad(output[:, 1], (0, pad)).reshape(ROWS, 128)
    rs = (ROWS, 128)
    loss = _final_loss(o0, o1, pa0.reshape(rs), pa1.reshape(rs),
                       pr0.reshape(rs), pr1.reshape(rs))
    return loss[0, 0]


# speedup vs baseline: 33.5079x; 7.3809x over previous
import jax
import jax.numpy as jnp
from jax import lax
from jax.experimental import pallas as pl
from jax.experimental.pallas import tpu as pltpu
from jax.experimental.pallas import tpu_sc as plsc

N_PAD = 50176
ROWS = N_PAD // 128
E_TOTAL = 1600000
BLK = 2048
NFULL = E_TOTAL // BLK
TAIL_BID = NFULL
TAIL_VALID_FROM = NFULL * BLK - (E_TOTAL - BLK)
NW = 32
KMAX = (NFULL + 1 + NW - 1) // NW
PER_TILE_SLICE = N_PAD // 16
DUMP_NODE = N_PAD - 1


def _node_xy_body(v_ref, th_ref, x_ref, y_ref):
    m = jnp.abs(v_ref[...])
    th = th_ref[...]
    x_ref[...] = m * jnp.cos(th)
    y_ref[...] = m * jnp.sin(th)


def _node_xy(v2, th2):
    return pl.pallas_call(
        _node_xy_body,
        out_shape=(
            jax.ShapeDtypeStruct((ROWS, 128), jnp.float32),
            jax.ShapeDtypeStruct((ROWS, 128), jnp.float32),
        ),
    )(v2, th2)


def _edge_body(x_hbm, y_hbm, from_hbm, to_hbm, a0_hbm, a1_hbm,
               pact0_hbm, pact1_hbm, prea0_hbm, prea1_hbm,
               xv, yv, fiv, tiv, a0v, a1v, actv, reav, acc_act, acc_rea):
    cid = lax.axis_index("c")
    sid = lax.axis_index("s")
    wid = cid * 16 + sid

    def _z(i, c):
        xv[pl.ds(i * 16, 16)] = jnp.zeros((16,), jnp.float32)
        return c
    lax.fori_loop(0, PER_TILE_SLICE // 16, _z, 0)
    pltpu.sync_copy(xv.at[pl.ds(0, PER_TILE_SLICE)],
                    acc_act.at[pl.ds(sid * PER_TILE_SLICE, PER_TILE_SLICE)])
    pltpu.sync_copy(xv.at[pl.ds(0, PER_TILE_SLICE)],
                    acc_rea.at[pl.ds(sid * PER_TILE_SLICE, PER_TILE_SLICE)])

    pltpu.sync_copy(x_hbm, xv)
    pltpu.sync_copy(y_hbm, yv)
    plsc.subcore_barrier()

    lanes = lax.iota(jnp.int32, 16)

    def _process_block(base, tail):
        pltpu.sync_copy(from_hbm.at[pl.ds(base, BLK)], fiv)
        pltpu.sync_copy(to_hbm.at[pl.ds(base, BLK)], tiv)
        pltpu.sync_copy(a0_hbm.at[pl.ds(base, BLK)], a0v)
        pltpu.sync_copy(a1_hbm.at[pl.ds(base, BLK)], a1v)

        def _vec(j, c):
            fi = fiv[pl.ds(j * 16, 16)]
            ti = tiv[pl.ds(j * 16, 16)]
            if tail:
                ok = (j * 16 + lanes) >= TAIL_VALID_FROM
                fi = jnp.where(ok, fi, DUMP_NODE)
                fiv[pl.ds(j * 16, 16)] = fi
            xf = plsc.load_gather(xv, [fi])
            yf = plsc.load_gather(yv, [fi])
            xt = plsc.load_gather(xv, [ti])
            yt = plsc.load_gather(yv, [ti])
            a0 = a0v[pl.ds(j * 16, 16)]
            a1 = a1v[pl.ds(j * 16, 16)]
            p = xf * xt + yf * yt
            q = yf * xt - xf * yt
            act = a0 * p + a1 * q
            rea = a0 * q - a1 * p
            if tail:
                act = jnp.where(ok, act, 0.0)
                rea = jnp.where(ok, rea, 0.0)
            actv[pl.ds(j * 16, 16)] = act
            reav[pl.ds(j * 16, 16)] = rea
            return c
        lax.fori_loop(0, BLK // 16, _vec, 0)

        pltpu.sync_copy(actv, acc_act.at[fiv], add=True)
        pltpu.sync_copy(reav, acc_rea.at[fiv], add=True)

    for k in range(KMAX):
        bid = wid + NW * k
        if k < KMAX - 1:
            _process_block(bid * BLK, tail=False)
        else:
            @pl.when(bid < NFULL)
            def _():
                _process_block(bid * BLK, tail=False)

            @pl.when(bid == TAIL_BID)
            def _():
                _process_block(E_TOTAL - BLK, tail=True)

    plsc.subcore_barrier()

    sl = pl.ds(sid * PER_TILE_SLICE, PER_TILE_SLICE)
    tsl = pl.ds(0, PER_TILE_SLICE)
    pltpu.sync_copy(acc_act.at[sl], xv.at[tsl])
    pltpu.sync_copy(acc_rea.at[sl], yv.at[tsl])

    @pl.when(cid == 0)
    def _():
        pltpu.sync_copy(xv.at[tsl], pact0_hbm.at[sl])
        pltpu.sync_copy(yv.at[tsl], prea0_hbm.at[sl])

    @pl.when(cid == 1)
    def _():
        pltpu.sync_copy(xv.at[tsl], pact1_hbm.at[sl])
        pltpu.sync_copy(yv.at[tsl], prea1_hbm.at[sl])


def _edge_accumulate(x1, y1, from1, to1, a0c, a1c):
    mesh = plsc.VectorSubcoreMesh(core_axis_name="c", subcore_axis_name="s")
    f = pl.kernel(
        _edge_body,
        out_type=tuple(
            jax.ShapeDtypeStruct((N_PAD,), jnp.float32) for _ in range(4)),
        mesh=mesh,
        compiler_params=pltpu.CompilerParams(needs_layout_passes=False),
        scratch_types=[
            pltpu.VMEM((N_PAD,), jnp.float32),
            pltpu.VMEM((N_PAD,), jnp.float32),
            pltpu.VMEM((BLK,), jnp.int32),
            pltpu.VMEM((BLK,), jnp.int32),
            pltpu.VMEM((BLK,), jnp.float32),
            pltpu.VMEM((BLK,), jnp.float32),
            pltpu.VMEM((BLK,), jnp.float32),
            pltpu.VMEM((BLK,), jnp.float32),
            pltpu.VMEM_SHARED((N_PAD,), jnp.float32),
            pltpu.VMEM_SHARED((N_PAD,), jnp.float32),
        ],
    )
    return f(x1, y1, from1, to1, a0c, a1c)


def _loss_body(o0_ref, o1_ref, a0_ref, a1_ref, r0_ref, r1_ref, out_ref):
    a = a0_ref[...] + a1_ref[...]
    r = r0_ref[...] + r1_ref[...]
    out_ref[0, 0] = jnp.sum(jnp.abs(o0_ref[...] - a) + jnp.abs(o1_ref[...] - r))


def _final_loss(o0, o1, a0, a1, r0, r1):
    return pl.pallas_call(
        _loss_body,
        out_shape=jax.ShapeDtypeStruct((1, 1), jnp.float32),
        out_specs=pl.BlockSpec(memory_space=pltpu.SMEM),
    )(o0, o1, a0, a1, r0, r1)


@jax.jit
def kernel(inputs, output, edges, attributes):
    del inputs
    n = output.shape[0]
    pad = N_PAD - n
    v2 = jnp.pad(output[:, 2], (0, pad)).reshape(ROWS, 128)
    th2 = jnp.pad(output[:, 3], (0, pad)).reshape(ROWS, 128)
    x2, y2 = _node_xy(v2, th2)

    from1 = edges[0].astype(jnp.int32)
    to1 = edges[1].astype(jnp.int32)
    pa0, pa1, pr0, pr1 = _edge_accumulate(
        x2.reshape(-1), y2.reshape(-1), from1, to1,
        attributes[:, 0], attributes[:, 1])

    o0 = jnp.pad(output[:, 0], (0, pad)).reshape(ROWS, 128)
    o1 = jnp.pad(output[:, 1], (0, pad)).reshape(ROWS, 128)
    rs = (ROWS, 128)
    loss = _final_loss(o0, o1, pa0.reshape(rs), pa1.reshape(rs),
                       pr0.reshape(rs), pr1.reshape(rs))
    return loss[0, 0]


# speedup vs baseline: 46.0775x; 1.3751x over previous
import jax
import jax.numpy as jnp
from jax import lax
from jax.experimental import pallas as pl
from jax.experimental.pallas import tpu as pltpu
from jax.experimental.pallas import tpu_sc as plsc

N_PAD = 50176
ROWS = N_PAD // 128
E_TOTAL = 1600000
BLK = 1024
NFULL = E_TOTAL // BLK
TAIL_BID = NFULL
TAIL_VALID_FROM = NFULL * BLK - (E_TOTAL - BLK)
NW = 32
KMAX = (NFULL + 1 + NW - 1) // NW
NVALID_ALL = NFULL + 1
PER_TILE_SLICE = N_PAD // 16
DUMP_NODE = N_PAD - 1


def _node_xy_body(v_ref, th_ref, x_ref, y_ref):
    m = jnp.abs(v_ref[...])
    th = th_ref[...]
    x_ref[...] = m * jnp.cos(th)
    y_ref[...] = m * jnp.sin(th)


def _node_xy(v2, th2):
    return pl.pallas_call(
        _node_xy_body,
        out_shape=(
            jax.ShapeDtypeStruct((ROWS, 128), jnp.float32),
            jax.ShapeDtypeStruct((ROWS, 128), jnp.float32),
        ),
    )(v2, th2)


def _edge_body(x_hbm, y_hbm, from_hbm, to_hbm, a0_hbm, a1_hbm,
               pact0_hbm, pact1_hbm, prea0_hbm, prea1_hbm,
               xv, yv, fiv0, fiv1, fiv2, tiv0, tiv1, a0v0, a0v1,
               a1v0, a1v1, actv0, actv1, reav0, reav1,
               in_sem0, in_sem1, sc_sem0, sc_sem1,
               acc_act, acc_rea):
    fiv = [fiv0, fiv1, fiv2]
    tiv = [tiv0, tiv1]
    a0v = [a0v0, a0v1]
    a1v = [a1v0, a1v1]
    actv = [actv0, actv1]
    reav = [reav0, reav1]
    in_sem = [in_sem0, in_sem1]
    sc_sem = [sc_sem0, sc_sem1]

    cid = lax.axis_index("c")
    sid = lax.axis_index("s")
    wid = cid * 16 + sid

    def _z(i, c):
        xv[pl.ds(i * 16, 16)] = jnp.zeros((16,), jnp.float32)
        return c
    lax.fori_loop(0, PER_TILE_SLICE // 16, _z, 0)
    pltpu.sync_copy(xv.at[pl.ds(0, PER_TILE_SLICE)],
                    acc_act.at[pl.ds(sid * PER_TILE_SLICE, PER_TILE_SLICE)])
    pltpu.sync_copy(xv.at[pl.ds(0, PER_TILE_SLICE)],
                    acc_rea.at[pl.ds(sid * PER_TILE_SLICE, PER_TILE_SLICE)])

    lanes = lax.iota(jnp.int32, 16)

    def _base(k):
        bid = wid + NW * k
        return jnp.minimum(bid * BLK, E_TOTAL - BLK)

    def _fire_inputs(k):
        b = _base(k)
        s = in_sem[k % 2]
        pltpu.async_copy(from_hbm.at[pl.ds(b, BLK)], fiv[k % 3], s)
        pltpu.async_copy(to_hbm.at[pl.ds(b, BLK)], tiv[k % 2], s)
        pltpu.async_copy(a0_hbm.at[pl.ds(b, BLK)], a0v[k % 2], s)
        pltpu.async_copy(a1_hbm.at[pl.ds(b, BLK)], a1v[k % 2], s)

    def _wait_inputs(k):
        s = in_sem[k % 2]
        pltpu.make_async_copy(from_hbm.at[pl.ds(0, BLK)], fiv[k % 3], s).wait()
        pltpu.make_async_copy(to_hbm.at[pl.ds(0, BLK)], tiv[k % 2], s).wait()
        pltpu.make_async_copy(a0_hbm.at[pl.ds(0, BLK)], a0v[k % 2], s).wait()
        pltpu.make_async_copy(a1_hbm.at[pl.ds(0, BLK)], a1v[k % 2], s).wait()

    def _fire_scatter(k):
        s = sc_sem[k % 2]
        pltpu.async_copy(actv[k % 2], acc_act.at[fiv[k % 3]], s, add=True)
        pltpu.async_copy(reav[k % 2], acc_rea.at[fiv[k % 3]], s, add=True)

    def _wait_scatter(k):
        s = sc_sem[k % 2]
        pltpu.make_async_copy(actv[k % 2], acc_act.at[fiv[k % 3]], s).wait()
        pltpu.make_async_copy(reav[k % 2], acc_rea.at[fiv[k % 3]], s).wait()

    def _compute(k, tail):
        fv, tv, av0, av1 = fiv[k % 3], tiv[k % 2], a0v[k % 2], a1v[k % 2]
        ov, rv = actv[k % 2], reav[k % 2]

        def _one(j):
            fi = fv[pl.ds(j * 16, 16)]
            ti = tv[pl.ds(j * 16, 16)]
            if tail:
                ok = (j * 16 + lanes) >= TAIL_VALID_FROM
                fi = jnp.where(ok, fi, DUMP_NODE)
                fv[pl.ds(j * 16, 16)] = fi
            xf = plsc.load_gather(xv, [fi])
            yf = plsc.load_gather(yv, [fi])
            xt = plsc.load_gather(xv, [ti])
            yt = plsc.load_gather(yv, [ti])
            a0 = av0[pl.ds(j * 16, 16)]
            a1 = av1[pl.ds(j * 16, 16)]
            p = xf * xt + yf * yt
            q = yf * xt - xf * yt
            act = a0 * p + a1 * q
            rea = a0 * q - a1 * p
            if tail:
                act = jnp.where(ok, act, 0.0)
                rea = jnp.where(ok, rea, 0.0)
            ov[pl.ds(j * 16, 16)] = act
            rv[pl.ds(j * 16, 16)] = rea

        def _vec(j2, c):
            _one(j2 * 2)
            _one(j2 * 2 + 1)
            return c
        lax.fori_loop(0, BLK // 32, _vec, 0)

    _fire_inputs(0)
    pltpu.sync_copy(x_hbm, xv)
    pltpu.sync_copy(y_hbm, yv)
    plsc.subcore_barrier()

    for k in range(KMAX):
        last = k == KMAX - 1
        bid = wid + NW * k
        valid = bid < NVALID_ALL

        if not last:
            _wait_inputs(k)
            if k >= 2:
                _wait_scatter(k - 2)
            if k + 1 < KMAX - 1:
                _fire_inputs(k + 1)
            else:
                @pl.when(wid + NW * (KMAX - 1) < NVALID_ALL)
                def _():
                    _fire_inputs(KMAX - 1)
            _compute(k, tail=False)
            _fire_scatter(k)
        else:
            @pl.when(valid)
            def _():
                _wait_inputs(k)
            _wait_scatter(k - 2)

            @pl.when(bid < NFULL)
            def _():
                _compute(k, tail=False)

            @pl.when(bid == TAIL_BID)
            def _():
                _compute(k, tail=True)

            @pl.when(valid)
            def _():
                _fire_scatter(k)

    _wait_scatter(KMAX - 2)

    @pl.when(wid + NW * (KMAX - 1) < NVALID_ALL)
    def _():
        _wait_scatter(KMAX - 1)

    plsc.subcore_barrier()

    sl = pl.ds(sid * PER_TILE_SLICE, PER_TILE_SLICE)
    tsl = pl.ds(0, PER_TILE_SLICE)
    pltpu.sync_copy(acc_act.at[sl], xv.at[tsl])
    pltpu.sync_copy(acc_rea.at[sl], yv.at[tsl])

    @pl.when(cid == 0)
    def _():
        pltpu.sync_copy(xv.at[tsl], pact0_hbm.at[sl])
        pltpu.sync_copy(yv.at[tsl], prea0_hbm.at[sl])

    @pl.when(cid == 1)
    def _():
        pltpu.sync_copy(xv.at[tsl], pact1_hbm.at[sl])
        pltpu.sync_copy(yv.at[tsl], prea1_hbm.at[sl])


def _edge_accumulate(x1, y1, from1, to1, a0c, a1c):
    mesh = plsc.VectorSubcoreMesh(core_axis_name="c", subcore_axis_name="s")
    f = pl.kernel(
        _edge_body,
        out_type=tuple(
            jax.ShapeDtypeStruct((N_PAD,), jnp.float32) for _ in range(4)),
        mesh=mesh,
        compiler_params=pltpu.CompilerParams(needs_layout_passes=False),
        scratch_types=[
            pltpu.VMEM((N_PAD,), jnp.float32),
            pltpu.VMEM((N_PAD,), jnp.float32),
            pltpu.VMEM((BLK,), jnp.int32),
            pltpu.VMEM((BLK,), jnp.int32),
            pltpu.VMEM((BLK,), jnp.int32),
            pltpu.VMEM((BLK,), jnp.int32),
            pltpu.VMEM((BLK,), jnp.int32),
            pltpu.VMEM((BLK,), jnp.float32),
            pltpu.VMEM((BLK,), jnp.float32),
            pltpu.VMEM((BLK,), jnp.float32),
            pltpu.VMEM((BLK,), jnp.float32),
            pltpu.VMEM((BLK,), jnp.float32),
            pltpu.VMEM((BLK,), jnp.float32),
            pltpu.VMEM((BLK,), jnp.float32),
            pltpu.VMEM((BLK,), jnp.float32),
            pltpu.SemaphoreType.DMA,
            pltpu.SemaphoreType.DMA,
            pltpu.SemaphoreType.DMA,
            pltpu.SemaphoreType.DMA,
            pltpu.VMEM_SHARED((N_PAD,), jnp.float32),
            pltpu.VMEM_SHARED((N_PAD,), jnp.float32),
        ],
    )
    return f(x1, y1, from1, to1, a0c, a1c)


def _loss_body(o0_ref, o1_ref, a0_ref, a1_ref, r0_ref, r1_ref, out_ref):
    a = a0_ref[...] + a1_ref[...]
    r = r0_ref[...] + r1_ref[...]
    out_ref[0, 0] = jnp.sum(jnp.abs(o0_ref[...] - a) + jnp.abs(o1_ref[...] - r))


def _final_loss(o0, o1, a0, a1, r0, r1):
    return pl.pallas_call(
        _loss_body,
        out_shape=jax.ShapeDtypeStruct((1, 1), jnp.float32),
        out_specs=pl.BlockSpec(memory_space=pltpu.SMEM),
    )(o0, o1, a0, a1, r0, r1)


@jax.jit
def kernel(inputs, output, edges, attributes):
    del inputs
    n = output.shape[0]
    pad = N_PAD - n
    v2 = jnp.pad(output[:, 2], (0, pad)).reshape(ROWS, 128)
    th2 = jnp.pad(output[:, 3], (0, pad)).reshape(ROWS, 128)
    x2, y2 = _node_xy(v2, th2)

    from1 = edges[0].astype(jnp.int32)
    to1 = edges[1].astype(jnp.int32)
    pa0, pa1, pr0, pr1 = _edge_accumulate(
        x2.reshape(-1), y2.reshape(-1), from1, to1,
        attributes[:, 0], attributes[:, 1])

    o0 = jnp.pad(output[:, 0], (0, pad)).reshape(ROWS, 128)
    o1 = jnp.pad(output[:, 1], (0, pad)).reshape(ROWS, 128)
    rs = (ROWS, 128)
    loss = _final_loss(o0, o1, pa0.reshape(rs), pa1.reshape(rs),
                       pr0.reshape(rs), pr1.reshape(rs))
    return loss[0, 0]
